# baseline (speedup 1.0000x reference)
import functools

import jax
import jax.numpy as jnp
from jax import lax
from jax.experimental import pallas as pl
from jax.experimental.pallas import tpu as pltpu
from jax.experimental.pallas import tpu_sc as plsc

N_NODES = 10000
N_EDGES = 320000
NODE_DIM = 128
EDGE_DIM = 16
HIDDEN_DIM = 128

NC = 2
NS = 16
NW = NC * NS
EPT = N_EDGES // NW
CHUNK = 80
NCHUNK = EPT // CHUNK
NBUF = 4
STRIPE_OFF = 624
STRIPE = 640

@functools.cache
def _sc_kernels():
    mesh = plsc.VectorSubcoreMesh(
        core_axis_name="c", subcore_axis_name="s",
        num_cores=NC, num_subcores=NS)

    @functools.partial(
        pl.kernel,
        mesh=mesh,
        out_type=(
            jax.ShapeDtypeStruct((N_EDGES, NODE_DIM // 2), jnp.int32),
            jax.ShapeDtypeStruct((N_EDGES, NODE_DIM // 2), jnp.int32),
        ),
        scratch_types=(
            [pltpu.VMEM((EPT,), jnp.int32)] * 2
            + [pltpu.VMEM((CHUNK, NODE_DIM // 2), jnp.int32)] * (2 * NBUF)
            + [pltpu.SemaphoreType.DMA] * (2 * NBUF)
        ),
        compiler_params=pltpu.CompilerParams(use_tc_tiling_on_sc=False),
    )
    def gather_k(x_hbm, src_hbm, dst_hbm, xi_hbm, xj_hbm, idxd, idxs, *rest):
        rows_a = rest[0:NBUF]
        rows_b = rest[NBUF:2 * NBUF]
        gsem = rest[2 * NBUF:3 * NBUF]
        wsem = rest[3 * NBUF:4 * NBUF]
        wid = lax.axis_index("s") * NC + lax.axis_index("c")
        base = wid * EPT
        pltpu.sync_copy(dst_hbm.at[pl.ds(base, EPT)], idxd)
        pltpu.sync_copy(src_hbm.at[pl.ds(base, EPT)], idxs)

        def issue_gather(c, b):
            off = pl.multiple_of(c * CHUNK, 8)
            pltpu.async_copy(x_hbm.at[idxd.at[pl.ds(off, CHUNK)]],
                             rows_a[b], gsem[b])
            pltpu.async_copy(x_hbm.at[idxs.at[pl.ds(off, CHUNK)]],
                             rows_b[b], gsem[b])

        def drain_gather(b):
            pltpu.make_async_copy(x_hbm.at[pl.ds(0, CHUNK)], rows_a[b],
                                  gsem[b]).wait()
            pltpu.make_async_copy(x_hbm.at[pl.ds(0, CHUNK)], rows_b[b],
                                  gsem[b]).wait()

        def issue_wb(c, b):
            off = pl.multiple_of(base + c * CHUNK, 8)
            wa = pltpu.async_copy(rows_a[b], xi_hbm.at[pl.ds(off, CHUNK)],
                                  wsem[b])
            wb = pltpu.async_copy(rows_b[b], xj_hbm.at[pl.ds(off, CHUNK)],
                                  wsem[b])
            return wa, wb

        def drain_wb(b):
            pltpu.make_async_copy(rows_a[b], xi_hbm.at[pl.ds(0, CHUNK)],
                                  wsem[b]).wait()
            pltpu.make_async_copy(rows_b[b], xj_hbm.at[pl.ds(0, CHUNK)],
                                  wsem[b]).wait()

        for b in range(NBUF):
            issue_gather(b, b)

        def body(k, carry):
            for b in range(NBUF):
                drain_gather(b)
                issue_wb(k * NBUF + b, b)
            for b in range(NBUF):
                c2 = (k + 1) * NBUF + b

                @pl.when(c2 < NCHUNK)
                def _():
                    drain_wb(b)
                    issue_gather(c2, b)

            return carry

        nmain = NCHUNK // NBUF
        lax.fori_loop(0, nmain, body, 0)
        for b in range(1, NBUF):
            drain_wb(b)
        drain_gather(0)
        issue_wb(NCHUNK - 1, 0)
        drain_wb(0)

    @functools.partial(
        pl.kernel,
        mesh=mesh,
        out_type=jax.ShapeDtypeStruct((NC, N_NODES, NODE_DIM), jnp.float32),
        scratch_types=(
            [pltpu.VMEM((EPT,), jnp.int32)]
            + [pltpu.VMEM((CHUNK, NODE_DIM), jnp.float32)] * NBUF
            + [pltpu.SemaphoreType.DMA] * (2 * NBUF)
            + [pltpu.VMEM_SHARED((N_NODES, NODE_DIM), jnp.float32)]
        ),
    )
    def scatter_k(msg_hbm, dst_hbm, zeros_hbm, out_hbm, idx_v, *rest):
        rows = rest[0:NBUF]
        gsem = rest[NBUF:2 * NBUF]
        ssem = rest[2 * NBUF:3 * NBUF]
        acc_sh = rest[3 * NBUF]
        cid = lax.axis_index("c")
        sid = lax.axis_index("s")
        wid = sid * NC + cid
        base = wid * EPT

        pltpu.sync_copy(zeros_hbm.at[pl.ds(sid * STRIPE_OFF, STRIPE)],
                        acc_sh.at[pl.ds(sid * STRIPE_OFF, STRIPE)])
        pltpu.sync_copy(dst_hbm.at[pl.ds(base, EPT)], idx_v)
        plsc.subcore_barrier()

        def issue_load(c, b):
            off = pl.multiple_of(base + c * CHUNK, 8)
            pltpu.async_copy(msg_hbm.at[pl.ds(off, CHUNK)], rows[b], gsem[b])

        def drain_load(b):
            pltpu.make_async_copy(msg_hbm.at[pl.ds(0, CHUNK)], rows[b],
                                  gsem[b]).wait()

        def issue_scat(c, b):
            off = pl.multiple_of(c * CHUNK, 8)
            pltpu.async_copy(rows[b], acc_sh.at[idx_v.at[pl.ds(off, CHUNK)]],
                             ssem[b], add=True)

        def drain_scat(b):
            pltpu.make_async_copy(rows[b], acc_sh.at[pl.ds(0, CHUNK)],
                                  ssem[b]).wait()

        for b in range(NBUF):
            issue_load(b, b)

        def body(k, carry):
            for b in range(NBUF):
                drain_load(b)
                issue_scat(k * NBUF + b, b)
            for b in range(NBUF):
                c2 = (k + 1) * NBUF + b

                @pl.when(c2 < NCHUNK)
                def _():
                    drain_scat(b)
                    issue_load(c2, b)

            return carry

        lax.fori_loop(0, NCHUNK // NBUF, body, 0)
        for b in range(1, NBUF):
            drain_scat(b)
        drain_load(0)
        issue_scat(NCHUNK - 1, 0)
        drain_scat(0)
        plsc.subcore_barrier()
        pltpu.sync_copy(acc_sh.at[pl.ds(sid * STRIPE_OFF, STRIPE)],
                        out_hbm.at[cid, pl.ds(sid * STRIPE_OFF, STRIPE)])

    return gather_k, scatter_k


def _mlp_body(xi_ref, xj_ref, e_ref, w1a_ref, w1b_ref, w1c_ref, b1_ref,
              w2_ref, b2_ref, w3_ref, b3_ref, out_ref):
    f32 = jnp.float32
    bf = jnp.bfloat16
    h = jnp.dot(xi_ref[...], w1a_ref[...], preferred_element_type=f32)
    h += jnp.dot(xj_ref[...], w1b_ref[...], preferred_element_type=f32)
    h += jnp.dot(e_ref[...], w1c_ref[...], preferred_element_type=f32)
    h = jnp.maximum(h + b1_ref[...], 0.0).astype(bf)
    h = jnp.maximum(
        jnp.dot(h, w2_ref[...], preferred_element_type=f32) + b2_ref[...],
        0.0).astype(bf)
    out_ref[...] = jnp.dot(h, w3_ref[...], preferred_element_type=f32) + b3_ref[...]


def _run_mlp(xi, xj, e, w1a, w1b, w1c, b1, w2, b2, w3, b3, block_e):
    grid = N_EDGES // block_e
    full = lambda s: pl.BlockSpec(s, lambda i: (0, 0))
    return pl.pallas_call(
        _mlp_body,
        grid=(grid,),
        in_specs=[
            pl.BlockSpec((block_e, NODE_DIM), lambda i: (i, 0)),
            pl.BlockSpec((block_e, NODE_DIM), lambda i: (i, 0)),
            pl.BlockSpec((block_e, EDGE_DIM), lambda i: (i, 0)),
            full((NODE_DIM, HIDDEN_DIM)),
            full((NODE_DIM, HIDDEN_DIM)),
            full((EDGE_DIM, HIDDEN_DIM)),
            full((1, HIDDEN_DIM)),
            full((HIDDEN_DIM, HIDDEN_DIM)),
            full((1, HIDDEN_DIM)),
            full((HIDDEN_DIM, NODE_DIM)),
            full((1, NODE_DIM)),
        ],
        out_specs=pl.BlockSpec((block_e, NODE_DIM), lambda i: (i, 0)),
        out_shape=jax.ShapeDtypeStruct((N_EDGES, NODE_DIM), jnp.float32),
    )(xi, xj, e, w1a, w1b, w1c, b1, w2, b2, w3, b3)


def _update_body(x_ref, p_ref, u1a_ref, u1b_ref, b1_ref, w2_ref, b2_ref,
                 gamma_ref, beta_ref, out_ref):
    f32 = jnp.float32
    xv = x_ref[...]
    aggr = p_ref[0] + p_ref[1]
    h = jnp.dot(xv, u1a_ref[...], preferred_element_type=f32)
    h += jnp.dot(aggr, u1b_ref[...], preferred_element_type=f32)
    h = jnp.maximum(h + b1_ref[...], 0.0)
    o = jnp.dot(h, w2_ref[...], preferred_element_type=f32) + b2_ref[...]
    y = xv + o
    mu = jnp.mean(y, axis=-1, keepdims=True)
    var = jnp.mean((y - mu) ** 2, axis=-1, keepdims=True)
    out_ref[...] = (y - mu) * lax.rsqrt(var + 1e-5) * gamma_ref[...] + beta_ref[...]


def _run_update(x, partials, u1a, u1b, b1, w2, b2, gamma, beta, block_n):
    grid = N_NODES // block_n
    full = lambda s: pl.BlockSpec(s, lambda i: (0,) * len(s))
    return pl.pallas_call(
        _update_body,
        grid=(grid,),
        in_specs=[
            pl.BlockSpec((block_n, NODE_DIM), lambda i: (i, 0)),
            pl.BlockSpec((NC, block_n, NODE_DIM), lambda i: (0, i, 0)),
            full((NODE_DIM, HIDDEN_DIM)),
            full((NODE_DIM, HIDDEN_DIM)),
            full((1, HIDDEN_DIM)),
            full((HIDDEN_DIM, NODE_DIM)),
            full((1, NODE_DIM)),
            full((1, NODE_DIM)),
            full((1, NODE_DIM)),
        ],
        out_specs=pl.BlockSpec((block_n, NODE_DIM), lambda i: (i, 0)),
        out_shape=jax.ShapeDtypeStruct((N_NODES, NODE_DIM), jnp.float32),
    )(x, partials, u1a, u1b, b1, w2, b2, gamma, beta)


def kernel(x, edge_index, edge_attr, m_w1, m_b1, m_w2, m_b2, m_w3, m_b3,
           u_w1, u_b1, u_w2, u_b2, gamma, beta):
    src = edge_index[0].astype(jnp.int32)
    dst = edge_index[1].astype(jnp.int32)

    gather_k, scatter_k = _sc_kernels()
    bf = jnp.bfloat16
    x32 = lax.bitcast_convert_type(
        x.astype(bf).reshape(N_NODES, NODE_DIM // 2, 2), jnp.int32)
    xi32, xj32 = gather_k(x32, src, dst)
    xi = lax.bitcast_convert_type(xi32, bf).reshape(N_EDGES, NODE_DIM)
    xj = lax.bitcast_convert_type(xj32, bf).reshape(N_EDGES, NODE_DIM)

    w1a = m_w1[:NODE_DIM].astype(bf)
    w1b = m_w1[NODE_DIM:2 * NODE_DIM].astype(bf)
    w1c = m_w1[2 * NODE_DIM:].astype(bf)
    m_w2 = m_w2.astype(bf)
    m_w3 = m_w3.astype(bf)
    edge_attr = edge_attr.astype(bf)
    msg = _run_mlp(xi, xj, edge_attr, w1a, w1b, w1c,
                   m_b1.reshape(1, -1), m_w2, m_b2.reshape(1, -1),
                   m_w3, m_b3.reshape(1, -1), block_e=2000)

    zeros = jnp.zeros((N_NODES, NODE_DIM), jnp.float32)
    partials = scatter_k(msg, dst, zeros)

    u1a = u_w1[:NODE_DIM]
    u1b = u_w1[NODE_DIM:]
    return _run_update(x, partials, u1a, u1b, u_b1.reshape(1, -1),
                       u_w2, u_b2.reshape(1, -1),
                       gamma.reshape(1, -1), beta.reshape(1, -1), block_n=1000)


# speedup vs baseline: 3.0472x; 3.0472x over previous
import functools

import jax
import jax.numpy as jnp
from jax import lax
from jax.experimental import pallas as pl
from jax.experimental.pallas import tpu as pltpu
from jax.experimental.pallas import tpu_sc as plsc

N_NODES = 10000
N_EDGES = 320000
NODE_DIM = 128
EDGE_DIM = 16
HIDDEN_DIM = 128

NC = 2
NS = 16
NW = NC * NS
EPT = N_EDGES // NW
CHUNK = 80
NCHUNK = EPT // CHUNK
NBUF = 4
STRIPE_OFF = 624
STRIPE = 640

@functools.cache
def _sc_kernels():
    mesh = plsc.VectorSubcoreMesh(
        core_axis_name="c", subcore_axis_name="s",
        num_cores=NC, num_subcores=NS)

    @functools.partial(
        pl.kernel,
        mesh=mesh,
        out_type=(
            jax.ShapeDtypeStruct((N_EDGES, NODE_DIM), jnp.float32),
            jax.ShapeDtypeStruct((N_EDGES, NODE_DIM), jnp.float32),
        ),
        scratch_types=(
            [pltpu.VMEM((EPT,), jnp.int32)] * 2
            + [pltpu.VMEM((CHUNK, NODE_DIM), jnp.float32)] * (2 * NBUF)
            + [pltpu.SemaphoreType.DMA] * (2 * NBUF)
        ),
    )
    def gather_k(x_hbm, src_hbm, dst_hbm, xi_hbm, xj_hbm, idxd, idxs, *rest):
        rows_a = rest[0:NBUF]
        rows_b = rest[NBUF:2 * NBUF]
        gsem = rest[2 * NBUF:3 * NBUF]
        wsem = rest[3 * NBUF:4 * NBUF]
        wid = lax.axis_index("s") * NC + lax.axis_index("c")
        base = wid * EPT
        pltpu.sync_copy(dst_hbm.at[pl.ds(base, EPT)], idxd)
        pltpu.sync_copy(src_hbm.at[pl.ds(base, EPT)], idxs)

        def issue_gather(c, b):
            off = pl.multiple_of(c * CHUNK, 8)
            pltpu.async_copy(x_hbm.at[idxd.at[pl.ds(off, CHUNK)]],
                             rows_a[b], gsem[b])
            pltpu.async_copy(x_hbm.at[idxs.at[pl.ds(off, CHUNK)]],
                             rows_b[b], gsem[b])

        def drain_gather(b):
            pltpu.make_async_copy(x_hbm.at[pl.ds(0, CHUNK)], rows_a[b],
                                  gsem[b]).wait()
            pltpu.make_async_copy(x_hbm.at[pl.ds(0, CHUNK)], rows_b[b],
                                  gsem[b]).wait()

        def issue_wb(c, b):
            off = pl.multiple_of(base + c * CHUNK, 8)
            wa = pltpu.async_copy(rows_a[b], xi_hbm.at[pl.ds(off, CHUNK)],
                                  wsem[b])
            wb = pltpu.async_copy(rows_b[b], xj_hbm.at[pl.ds(off, CHUNK)],
                                  wsem[b])
            return wa, wb

        def drain_wb(b):
            pltpu.make_async_copy(rows_a[b], xi_hbm.at[pl.ds(0, CHUNK)],
                                  wsem[b]).wait()
            pltpu.make_async_copy(rows_b[b], xj_hbm.at[pl.ds(0, CHUNK)],
                                  wsem[b]).wait()

        for b in range(NBUF):
            issue_gather(b, b)

        def body(k, carry):
            for b in range(NBUF):
                drain_gather(b)
                issue_wb(k * NBUF + b, b)
            for b in range(NBUF):
                c2 = (k + 1) * NBUF + b

                @pl.when(c2 < NCHUNK)
                def _():
                    drain_wb(b)
                    issue_gather(c2, b)

            return carry

        nmain = NCHUNK // NBUF
        lax.fori_loop(0, nmain, body, 0)
        for b in range(1, NBUF):
            drain_wb(b)
        drain_gather(0)
        issue_wb(NCHUNK - 1, 0)
        drain_wb(0)

    @functools.partial(
        pl.kernel,
        mesh=mesh,
        out_type=jax.ShapeDtypeStruct((NC, N_NODES, NODE_DIM), jnp.float32),
        scratch_types=(
            [pltpu.VMEM((EPT,), jnp.int32)]
            + [pltpu.VMEM((CHUNK, NODE_DIM), jnp.float32)] * NBUF
            + [pltpu.SemaphoreType.DMA] * (2 * NBUF)
            + [pltpu.VMEM_SHARED((N_NODES, NODE_DIM), jnp.float32)]
        ),
    )
    def scatter_k(msg_hbm, dst_hbm, zeros_hbm, out_hbm, idx_v, *rest):
        rows = rest[0:NBUF]
        gsem = rest[NBUF:2 * NBUF]
        ssem = rest[2 * NBUF:3 * NBUF]
        acc_sh = rest[3 * NBUF]
        cid = lax.axis_index("c")
        sid = lax.axis_index("s")
        wid = sid * NC + cid
        base = wid * EPT

        pltpu.sync_copy(zeros_hbm.at[pl.ds(sid * STRIPE_OFF, STRIPE)],
                        acc_sh.at[pl.ds(sid * STRIPE_OFF, STRIPE)])
        pltpu.sync_copy(dst_hbm.at[pl.ds(base, EPT)], idx_v)
        plsc.subcore_barrier()

        def issue_load(c, b):
            off = pl.multiple_of(base + c * CHUNK, 8)
            pltpu.async_copy(msg_hbm.at[pl.ds(off, CHUNK)], rows[b], gsem[b])

        def drain_load(b):
            pltpu.make_async_copy(msg_hbm.at[pl.ds(0, CHUNK)], rows[b],
                                  gsem[b]).wait()

        def issue_scat(c, b):
            off = pl.multiple_of(c * CHUNK, 8)
            pltpu.async_copy(rows[b], acc_sh.at[idx_v.at[pl.ds(off, CHUNK)]],
                             ssem[b], add=True)

        def drain_scat(b):
            pltpu.make_async_copy(rows[b], acc_sh.at[pl.ds(0, CHUNK)],
                                  ssem[b]).wait()

        for b in range(NBUF):
            issue_load(b, b)

        def body(k, carry):
            for b in range(NBUF):
                drain_load(b)
                issue_scat(k * NBUF + b, b)
            for b in range(NBUF):
                c2 = (k + 1) * NBUF + b

                @pl.when(c2 < NCHUNK)
                def _():
                    drain_scat(b)
                    issue_load(c2, b)

            return carry

        lax.fori_loop(0, NCHUNK // NBUF, body, 0)
        for b in range(1, NBUF):
            drain_scat(b)
        drain_load(0)
        issue_scat(NCHUNK - 1, 0)
        drain_scat(0)
        plsc.subcore_barrier()
        pltpu.sync_copy(acc_sh.at[pl.ds(sid * STRIPE_OFF, STRIPE)],
                        out_hbm.at[cid, pl.ds(sid * STRIPE_OFF, STRIPE)])

    return gather_k, scatter_k


def _mlp_body(xi_ref, xj_ref, e_ref, w1a_ref, w1b_ref, w1c_ref, b1_ref,
              w2_ref, b2_ref, w3_ref, b3_ref, out_ref):
    f32 = jnp.float32
    bf = jnp.bfloat16
    h = jnp.dot(xi_ref[...].astype(bf), w1a_ref[...], preferred_element_type=f32)
    h += jnp.dot(xj_ref[...].astype(bf), w1b_ref[...], preferred_element_type=f32)
    h += jnp.dot(e_ref[...], w1c_ref[...], preferred_element_type=f32)
    h = jnp.maximum(h + b1_ref[...], 0.0).astype(bf)
    h = jnp.maximum(
        jnp.dot(h, w2_ref[...], preferred_element_type=f32) + b2_ref[...],
        0.0).astype(bf)
    out_ref[...] = jnp.dot(h, w3_ref[...], preferred_element_type=f32) + b3_ref[...]


def _run_mlp(xi, xj, e, w1a, w1b, w1c, b1, w2, b2, w3, b3, block_e):
    grid = N_EDGES // block_e
    full = lambda s: pl.BlockSpec(s, lambda i: (0, 0))
    return pl.pallas_call(
        _mlp_body,
        grid=(grid,),
        in_specs=[
            pl.BlockSpec((block_e, NODE_DIM), lambda i: (i, 0)),
            pl.BlockSpec((block_e, NODE_DIM), lambda i: (i, 0)),
            pl.BlockSpec((block_e, EDGE_DIM), lambda i: (i, 0)),
            full((NODE_DIM, HIDDEN_DIM)),
            full((NODE_DIM, HIDDEN_DIM)),
            full((EDGE_DIM, HIDDEN_DIM)),
            full((1, HIDDEN_DIM)),
            full((HIDDEN_DIM, HIDDEN_DIM)),
            full((1, HIDDEN_DIM)),
            full((HIDDEN_DIM, NODE_DIM)),
            full((1, NODE_DIM)),
        ],
        out_specs=pl.BlockSpec((block_e, NODE_DIM), lambda i: (i, 0)),
        out_shape=jax.ShapeDtypeStruct((N_EDGES, NODE_DIM), jnp.float32),
    )(xi, xj, e, w1a, w1b, w1c, b1, w2, b2, w3, b3)


def _update_body(x_ref, p_ref, u1a_ref, u1b_ref, b1_ref, w2_ref, b2_ref,
                 gamma_ref, beta_ref, out_ref):
    f32 = jnp.float32
    xv = x_ref[...]
    aggr = p_ref[0] + p_ref[1]
    h = jnp.dot(xv, u1a_ref[...], preferred_element_type=f32)
    h += jnp.dot(aggr, u1b_ref[...], preferred_element_type=f32)
    h = jnp.maximum(h + b1_ref[...], 0.0)
    o = jnp.dot(h, w2_ref[...], preferred_element_type=f32) + b2_ref[...]
    y = xv + o
    mu = jnp.mean(y, axis=-1, keepdims=True)
    var = jnp.mean((y - mu) ** 2, axis=-1, keepdims=True)
    out_ref[...] = (y - mu) * lax.rsqrt(var + 1e-5) * gamma_ref[...] + beta_ref[...]


def _run_update(x, partials, u1a, u1b, b1, w2, b2, gamma, beta, block_n):
    grid = N_NODES // block_n
    full = lambda s: pl.BlockSpec(s, lambda i: (0,) * len(s))
    return pl.pallas_call(
        _update_body,
        grid=(grid,),
        in_specs=[
            pl.BlockSpec((block_n, NODE_DIM), lambda i: (i, 0)),
            pl.BlockSpec((NC, block_n, NODE_DIM), lambda i: (0, i, 0)),
            full((NODE_DIM, HIDDEN_DIM)),
            full((NODE_DIM, HIDDEN_DIM)),
            full((1, HIDDEN_DIM)),
            full((HIDDEN_DIM, NODE_DIM)),
            full((1, NODE_DIM)),
            full((1, NODE_DIM)),
            full((1, NODE_DIM)),
        ],
        out_specs=pl.BlockSpec((block_n, NODE_DIM), lambda i: (i, 0)),
        out_shape=jax.ShapeDtypeStruct((N_NODES, NODE_DIM), jnp.float32),
    )(x, partials, u1a, u1b, b1, w2, b2, gamma, beta)


def kernel(x, edge_index, edge_attr, m_w1, m_b1, m_w2, m_b2, m_w3, m_b3,
           u_w1, u_b1, u_w2, u_b2, gamma, beta):
    src = edge_index[0].astype(jnp.int32)
    dst = edge_index[1].astype(jnp.int32)

    gather_k, scatter_k = _sc_kernels()
    bf = jnp.bfloat16
    xi, xj = gather_k(x, src, dst)

    w1a = m_w1[:NODE_DIM].astype(bf)
    w1b = m_w1[NODE_DIM:2 * NODE_DIM].astype(bf)
    w1c = m_w1[2 * NODE_DIM:].astype(bf)
    m_w2 = m_w2.astype(bf)
    m_w3 = m_w3.astype(bf)
    edge_attr = edge_attr.astype(bf)
    msg = _run_mlp(xi, xj, edge_attr, w1a, w1b, w1c,
                   m_b1.reshape(1, -1), m_w2, m_b2.reshape(1, -1),
                   m_w3, m_b3.reshape(1, -1), block_e=2000)

    zeros = jnp.zeros((N_NODES, NODE_DIM), jnp.float32)
    partials = scatter_k(msg, dst, zeros)

    u1a = u_w1[:NODE_DIM]
    u1b = u_w1[NODE_DIM:]
    return _run_update(x, partials, u1a, u1b, u_b1.reshape(1, -1),
                       u_w2, u_b2.reshape(1, -1),
                       gamma.reshape(1, -1), beta.reshape(1, -1), block_n=1000)


# speedup vs baseline: 3.4489x; 1.1318x over previous
import functools

import jax
import jax.numpy as jnp
from jax import lax
from jax.experimental import pallas as pl
from jax.experimental.pallas import tpu as pltpu
from jax.experimental.pallas import tpu_sc as plsc

N_NODES = 10000
N_EDGES = 320000
NODE_DIM = 128
EDGE_DIM = 16
HIDDEN_DIM = 128

NC = 2
NS = 16
NW = NC * NS
EPT = N_EDGES // NW
CHUNK = 80
NCHUNK = EPT // CHUNK
NBUF = 4
STRIPE_OFF = 624
STRIPE = 640

@functools.cache
def _sc_kernels():
    mesh = plsc.VectorSubcoreMesh(
        core_axis_name="c", subcore_axis_name="s",
        num_cores=NC, num_subcores=NS)

    @functools.partial(
        pl.kernel,
        mesh=mesh,
        out_type=(
            jax.ShapeDtypeStruct((N_EDGES, NODE_DIM), jnp.float32),
            jax.ShapeDtypeStruct((N_EDGES, NODE_DIM), jnp.float32),
        ),
        scratch_types=(
            [pltpu.VMEM((EPT,), jnp.int32)] * 2
            + [pltpu.VMEM((CHUNK, NODE_DIM), jnp.float32)] * (2 * NBUF)
            + [pltpu.SemaphoreType.DMA] * (2 * NBUF)
        ),
    )
    def gather_k(x_hbm, src_hbm, dst_hbm, xi_hbm, xj_hbm, idxd, idxs, *rest):
        rows_a = rest[0:NBUF]
        rows_b = rest[NBUF:2 * NBUF]
        gsem = rest[2 * NBUF:3 * NBUF]
        wsem = rest[3 * NBUF:4 * NBUF]
        wid = lax.axis_index("s") * NC + lax.axis_index("c")
        base = wid * EPT
        pltpu.sync_copy(dst_hbm.at[pl.ds(base, EPT)], idxd)
        pltpu.sync_copy(src_hbm.at[pl.ds(base, EPT)], idxs)

        def issue_gather(c, b):
            off = pl.multiple_of(c * CHUNK, 8)
            pltpu.async_copy(x_hbm.at[idxd.at[pl.ds(off, CHUNK)]],
                             rows_a[b], gsem[b])
            pltpu.async_copy(x_hbm.at[idxs.at[pl.ds(off, CHUNK)]],
                             rows_b[b], gsem[b])

        def drain_gather(b):
            pltpu.make_async_copy(x_hbm.at[pl.ds(0, CHUNK)], rows_a[b],
                                  gsem[b]).wait()
            pltpu.make_async_copy(x_hbm.at[pl.ds(0, CHUNK)], rows_b[b],
                                  gsem[b]).wait()

        def issue_wb(c, b):
            off = pl.multiple_of(base + c * CHUNK, 8)
            wa = pltpu.async_copy(rows_a[b], xi_hbm.at[pl.ds(off, CHUNK)],
                                  wsem[b])
            wb = pltpu.async_copy(rows_b[b], xj_hbm.at[pl.ds(off, CHUNK)],
                                  wsem[b])
            return wa, wb

        def drain_wb(b):
            pltpu.make_async_copy(rows_a[b], xi_hbm.at[pl.ds(0, CHUNK)],
                                  wsem[b]).wait()
            pltpu.make_async_copy(rows_b[b], xj_hbm.at[pl.ds(0, CHUNK)],
                                  wsem[b]).wait()

        for b in range(NBUF):
            issue_gather(b, b)

        def body(k, carry):
            for b in range(NBUF):
                drain_gather(b)
                issue_wb(k * NBUF + b, b)
            for b in range(NBUF):
                c2 = (k + 1) * NBUF + b

                @pl.when(c2 < NCHUNK)
                def _():
                    drain_wb(b)
                    issue_gather(c2, b)

            return carry

        nmain = NCHUNK // NBUF
        lax.fori_loop(0, nmain, body, 0)
        for b in range(1, NBUF):
            drain_wb(b)
        drain_gather(0)
        issue_wb(NCHUNK - 1, 0)
        drain_wb(0)

    @functools.partial(
        pl.kernel,
        mesh=mesh,
        out_type=jax.ShapeDtypeStruct((NC, N_NODES, NODE_DIM), jnp.float32),
        scratch_types=(
            [pltpu.VMEM((EPT,), jnp.int32)]
            + [pltpu.VMEM((CHUNK, NODE_DIM), jnp.float32)] * NBUF
            + [pltpu.SemaphoreType.DMA] * (2 * NBUF)
            + [pltpu.VMEM_SHARED((N_NODES, NODE_DIM), jnp.float32)]
        ),
    )
    def scatter_k(msg_hbm, dst_hbm, zeros_hbm, out_hbm, idx_v, *rest):
        rows = rest[0:NBUF]
        gsem = rest[NBUF:2 * NBUF]
        ssem = rest[2 * NBUF:3 * NBUF]
        acc_sh = rest[3 * NBUF]
        cid = lax.axis_index("c")
        sid = lax.axis_index("s")
        wid = sid * NC + cid
        base = wid * EPT

        pltpu.sync_copy(zeros_hbm.at[pl.ds(sid * STRIPE_OFF, STRIPE)],
                        acc_sh.at[pl.ds(sid * STRIPE_OFF, STRIPE)])
        pltpu.sync_copy(dst_hbm.at[pl.ds(base, EPT)], idx_v)
        plsc.subcore_barrier()

        def issue_load(c, b):
            off = pl.multiple_of(base + c * CHUNK, 8)
            pltpu.async_copy(msg_hbm.at[pl.ds(off, CHUNK)], rows[b], gsem[b])

        def drain_load(b):
            pltpu.make_async_copy(msg_hbm.at[pl.ds(0, CHUNK)], rows[b],
                                  gsem[b]).wait()

        def issue_scat(c, b):
            off = pl.multiple_of(c * CHUNK, 8)
            pltpu.async_copy(rows[b], acc_sh.at[idx_v.at[pl.ds(off, CHUNK)]],
                             ssem[b], add=True)

        def drain_scat(b):
            pltpu.make_async_copy(rows[b], acc_sh.at[pl.ds(0, CHUNK)],
                                  ssem[b]).wait()

        for b in range(NBUF):
            issue_load(b, b)

        def body(k, carry):
            for b in range(NBUF):
                drain_load(b)
                issue_scat(k * NBUF + b, b)
            for b in range(NBUF):
                c2 = (k + 1) * NBUF + b

                @pl.when(c2 < NCHUNK)
                def _():
                    drain_scat(b)
                    issue_load(c2, b)

            return carry

        lax.fori_loop(0, NCHUNK // NBUF, body, 0)
        for b in range(1, NBUF):
            drain_scat(b)
        drain_load(0)
        issue_scat(NCHUNK - 1, 0)
        drain_scat(0)
        plsc.subcore_barrier()
        pltpu.sync_copy(acc_sh.at[pl.ds(sid * STRIPE_OFF, STRIPE)],
                        out_hbm.at[cid, pl.ds(sid * STRIPE_OFF, STRIPE)])

    return gather_k, scatter_k


def _mlp_body(xi_ref, xj_ref, e_ref, w1a_ref, w1b_ref, w1c_ref, b1_ref,
              w2_ref, b2_ref, w3_ref, b3_ref, out_ref):
    f32 = jnp.float32
    bf = jnp.bfloat16
    h = jnp.dot(xi_ref[...].astype(bf), w1a_ref[...], preferred_element_type=f32)
    h += jnp.dot(xj_ref[...].astype(bf), w1b_ref[...], preferred_element_type=f32)
    h += lax.dot_general(e_ref[...], w1c_ref[...], (((0,), (0,)), ((), ())),
                         preferred_element_type=f32)
    h = jnp.maximum(h + b1_ref[...], 0.0).astype(bf)
    h = jnp.maximum(
        jnp.dot(h, w2_ref[...], preferred_element_type=f32) + b2_ref[...],
        0.0).astype(bf)
    out_ref[...] = jnp.dot(h, w3_ref[...], preferred_element_type=f32) + b3_ref[...]


def _run_mlp(xi, xj, e, w1a, w1b, w1c, b1, w2, b2, w3, b3, block_e):
    grid = N_EDGES // block_e
    full = lambda s: pl.BlockSpec(s, lambda i: (0, 0))
    return pl.pallas_call(
        _mlp_body,
        grid=(grid,),
        in_specs=[
            pl.BlockSpec((block_e, NODE_DIM), lambda i: (i, 0)),
            pl.BlockSpec((block_e, NODE_DIM), lambda i: (i, 0)),
            pl.BlockSpec((EDGE_DIM, block_e), lambda i: (0, i)),
            full((NODE_DIM, HIDDEN_DIM)),
            full((NODE_DIM, HIDDEN_DIM)),
            full((EDGE_DIM, HIDDEN_DIM)),
            full((1, HIDDEN_DIM)),
            full((HIDDEN_DIM, HIDDEN_DIM)),
            full((1, HIDDEN_DIM)),
            full((HIDDEN_DIM, NODE_DIM)),
            full((1, NODE_DIM)),
        ],
        out_specs=pl.BlockSpec((block_e, NODE_DIM), lambda i: (i, 0)),
        out_shape=jax.ShapeDtypeStruct((N_EDGES, NODE_DIM), jnp.float32),
    )(xi, xj, e, w1a, w1b, w1c, b1, w2, b2, w3, b3)


def _update_body(x_ref, p_ref, u1a_ref, u1b_ref, b1_ref, w2_ref, b2_ref,
                 gamma_ref, beta_ref, out_ref):
    f32 = jnp.float32
    xv = x_ref[...]
    aggr = p_ref[0] + p_ref[1]
    h = jnp.dot(xv, u1a_ref[...], preferred_element_type=f32)
    h += jnp.dot(aggr, u1b_ref[...], preferred_element_type=f32)
    h = jnp.maximum(h + b1_ref[...], 0.0)
    o = jnp.dot(h, w2_ref[...], preferred_element_type=f32) + b2_ref[...]
    y = xv + o
    mu = jnp.mean(y, axis=-1, keepdims=True)
    var = jnp.mean((y - mu) ** 2, axis=-1, keepdims=True)
    out_ref[...] = (y - mu) * lax.rsqrt(var + 1e-5) * gamma_ref[...] + beta_ref[...]


def _run_update(x, partials, u1a, u1b, b1, w2, b2, gamma, beta, block_n):
    grid = N_NODES // block_n
    full = lambda s: pl.BlockSpec(s, lambda i: (0,) * len(s))
    return pl.pallas_call(
        _update_body,
        grid=(grid,),
        in_specs=[
            pl.BlockSpec((block_n, NODE_DIM), lambda i: (i, 0)),
            pl.BlockSpec((NC, block_n, NODE_DIM), lambda i: (0, i, 0)),
            full((NODE_DIM, HIDDEN_DIM)),
            full((NODE_DIM, HIDDEN_DIM)),
            full((1, HIDDEN_DIM)),
            full((HIDDEN_DIM, NODE_DIM)),
            full((1, NODE_DIM)),
            full((1, NODE_DIM)),
            full((1, NODE_DIM)),
        ],
        out_specs=pl.BlockSpec((block_n, NODE_DIM), lambda i: (i, 0)),
        out_shape=jax.ShapeDtypeStruct((N_NODES, NODE_DIM), jnp.float32),
    )(x, partials, u1a, u1b, b1, w2, b2, gamma, beta)


def kernel(x, edge_index, edge_attr, m_w1, m_b1, m_w2, m_b2, m_w3, m_b3,
           u_w1, u_b1, u_w2, u_b2, gamma, beta):
    src = edge_index[0].astype(jnp.int32)
    dst = edge_index[1].astype(jnp.int32)

    gather_k, scatter_k = _sc_kernels()
    bf = jnp.bfloat16
    xi, xj = gather_k(x, src, dst)

    w1a = m_w1[:NODE_DIM].astype(bf)
    w1b = m_w1[NODE_DIM:2 * NODE_DIM].astype(bf)
    w1c = m_w1[2 * NODE_DIM:].astype(bf)
    m_w2 = m_w2.astype(bf)
    m_w3 = m_w3.astype(bf)
    edge_attr = edge_attr.T.astype(bf)
    msg = _run_mlp(xi, xj, edge_attr, w1a, w1b, w1c,
                   m_b1.reshape(1, -1), m_w2, m_b2.reshape(1, -1),
                   m_w3, m_b3.reshape(1, -1), block_e=2560)

    zeros = jnp.zeros((N_NODES, NODE_DIM), jnp.float32)
    partials = scatter_k(msg, dst, zeros)

    u1a = u_w1[:NODE_DIM]
    u1b = u_w1[NODE_DIM:]
    return _run_update(x, partials, u1a, u1b, u_b1.reshape(1, -1),
                       u_w2, u_b2.reshape(1, -1),
                       gamma.reshape(1, -1), beta.reshape(1, -1), block_n=1000)


# speedup vs baseline: 3.9369x; 1.1415x over previous
import functools

import jax
import jax.numpy as jnp
from jax import lax
from jax.experimental import pallas as pl
from jax.experimental.pallas import tpu as pltpu
from jax.experimental.pallas import tpu_sc as plsc

N_NODES = 10000
N_EDGES = 320000
NODE_DIM = 128
EDGE_DIM = 16
HIDDEN_DIM = 128

NC = 2
NS = 16
NW = NC * NS
EPT = N_EDGES // NW
CHUNK = 80
NCHUNK = EPT // CHUNK
NBUF = 4
STRIPE_OFF = 624
STRIPE = 640

@functools.cache
def _sc_kernels():
    mesh = plsc.VectorSubcoreMesh(
        core_axis_name="c", subcore_axis_name="s",
        num_cores=NC, num_subcores=NS)

    @functools.partial(
        pl.kernel,
        mesh=mesh,
        out_type=jax.ShapeDtypeStruct((N_EDGES, NODE_DIM), jnp.float32),
        scratch_types=(
            [pltpu.VMEM((EPT,), jnp.int32)] * 2
            + [pltpu.VMEM((CHUNK, NODE_DIM), jnp.float32)] * (2 * NBUF)
            + [pltpu.SemaphoreType.DMA] * (2 * NBUF)
        ),
    )
    def gather_k(xa_hbm, xb_hbm, src_hbm, dst_hbm, g_hbm, idxd, idxs, *rest):
        rows_a = rest[0:NBUF]
        rows_b = rest[NBUF:2 * NBUF]
        gsem = rest[2 * NBUF:3 * NBUF]
        wsem = rest[3 * NBUF:4 * NBUF]
        wid = lax.axis_index("s") * NC + lax.axis_index("c")
        base = wid * EPT
        pltpu.sync_copy(dst_hbm.at[pl.ds(base, EPT)], idxd)
        pltpu.sync_copy(src_hbm.at[pl.ds(base, EPT)], idxs)

        def issue_gather(c, b):
            off = pl.multiple_of(c * CHUNK, 8)
            pltpu.async_copy(xa_hbm.at[idxd.at[pl.ds(off, CHUNK)]],
                             rows_a[b], gsem[b])
            pltpu.async_copy(xb_hbm.at[idxs.at[pl.ds(off, CHUNK)]],
                             rows_b[b], gsem[b])

        def drain_gather(b):
            pltpu.make_async_copy(xa_hbm.at[pl.ds(0, CHUNK)], rows_a[b],
                                  gsem[b]).wait()
            pltpu.make_async_copy(xa_hbm.at[pl.ds(0, CHUNK)], rows_b[b],
                                  gsem[b]).wait()

        def add_rows(b):
            ra, rb = rows_a[b], rows_b[b]

            def addrow(r, carry):
                for c8 in range(NODE_DIM // 16):
                    sl = pl.ds(c8 * 16, 16)
                    ra[r, sl] = ra[r, sl] + rb[r, sl]
                return carry

            lax.fori_loop(0, CHUNK, addrow, 0)

        def issue_wb(c, b):
            off = pl.multiple_of(base + c * CHUNK, 8)
            pltpu.async_copy(rows_a[b], g_hbm.at[pl.ds(off, CHUNK)], wsem[b])

        def drain_wb(b):
            pltpu.make_async_copy(rows_a[b], g_hbm.at[pl.ds(0, CHUNK)],
                                  wsem[b]).wait()

        for b in range(NBUF):
            issue_gather(b, b)

        def body(k, carry):
            for b in range(NBUF):
                drain_gather(b)
                add_rows(b)
                issue_wb(k * NBUF + b, b)
            for b in range(NBUF):
                c2 = (k + 1) * NBUF + b

                @pl.when(c2 < NCHUNK)
                def _():
                    drain_wb(b)
                    issue_gather(c2, b)

            return carry

        nmain = NCHUNK // NBUF
        lax.fori_loop(0, nmain, body, 0)
        for b in range(1, NBUF):
            drain_wb(b)
        drain_gather(0)
        add_rows(0)
        issue_wb(NCHUNK - 1, 0)
        drain_wb(0)

    @functools.partial(
        pl.kernel,
        mesh=mesh,
        out_type=jax.ShapeDtypeStruct((NC, N_NODES, NODE_DIM), jnp.float32),
        scratch_types=(
            [pltpu.VMEM((EPT,), jnp.int32)]
            + [pltpu.VMEM((CHUNK, NODE_DIM), jnp.float32)] * NBUF
            + [pltpu.SemaphoreType.DMA] * (2 * NBUF)
            + [pltpu.VMEM_SHARED((N_NODES, NODE_DIM), jnp.float32)]
        ),
    )
    def scatter_k(msg_hbm, dst_hbm, zeros_hbm, out_hbm, idx_v, *rest):
        rows = rest[0:NBUF]
        gsem = rest[NBUF:2 * NBUF]
        ssem = rest[2 * NBUF:3 * NBUF]
        acc_sh = rest[3 * NBUF]
        cid = lax.axis_index("c")
        sid = lax.axis_index("s")
        wid = sid * NC + cid
        base = wid * EPT

        pltpu.sync_copy(zeros_hbm.at[pl.ds(sid * STRIPE_OFF, STRIPE)],
                        acc_sh.at[pl.ds(sid * STRIPE_OFF, STRIPE)])
        pltpu.sync_copy(dst_hbm.at[pl.ds(base, EPT)], idx_v)
        plsc.subcore_barrier()

        def issue_load(c, b):
            off = pl.multiple_of(base + c * CHUNK, 8)
            pltpu.async_copy(msg_hbm.at[pl.ds(off, CHUNK)], rows[b], gsem[b])

        def drain_load(b):
            pltpu.make_async_copy(msg_hbm.at[pl.ds(0, CHUNK)], rows[b],
                                  gsem[b]).wait()

        def issue_scat(c, b):
            off = pl.multiple_of(c * CHUNK, 8)
            pltpu.async_copy(rows[b], acc_sh.at[idx_v.at[pl.ds(off, CHUNK)]],
                             ssem[b], add=True)

        def drain_scat(b):
            pltpu.make_async_copy(rows[b], acc_sh.at[pl.ds(0, CHUNK)],
                                  ssem[b]).wait()

        for b in range(NBUF):
            issue_load(b, b)

        def body(k, carry):
            for b in range(NBUF):
                drain_load(b)
                issue_scat(k * NBUF + b, b)
            for b in range(NBUF):
                c2 = (k + 1) * NBUF + b

                @pl.when(c2 < NCHUNK)
                def _():
                    drain_scat(b)
                    issue_load(c2, b)

            return carry

        lax.fori_loop(0, NCHUNK // NBUF, body, 0)
        for b in range(1, NBUF):
            drain_scat(b)
        drain_load(0)
        issue_scat(NCHUNK - 1, 0)
        drain_scat(0)
        plsc.subcore_barrier()
        pltpu.sync_copy(acc_sh.at[pl.ds(sid * STRIPE_OFF, STRIPE)],
                        out_hbm.at[cid, pl.ds(sid * STRIPE_OFF, STRIPE)])

    return gather_k, scatter_k


def _pre_body(x_ref, w1a_ref, w1b_ref, b1_ref, xa_ref, xb_ref):
    f32 = jnp.float32
    xb16 = x_ref[...].astype(jnp.bfloat16)
    xa_ref[...] = (jnp.dot(xb16, w1a_ref[...], preferred_element_type=f32)
                   + b1_ref[...])
    xb_ref[...] = jnp.dot(xb16, w1b_ref[...], preferred_element_type=f32)


def _run_pre(x, w1a, w1b, b1, block_n):
    grid = N_NODES // block_n
    full = lambda s: pl.BlockSpec(s, lambda i: (0, 0))
    return pl.pallas_call(
        _pre_body,
        grid=(grid,),
        in_specs=[
            pl.BlockSpec((block_n, NODE_DIM), lambda i: (i, 0)),
            full((NODE_DIM, HIDDEN_DIM)),
            full((NODE_DIM, HIDDEN_DIM)),
            full((1, HIDDEN_DIM)),
        ],
        out_specs=[pl.BlockSpec((block_n, HIDDEN_DIM), lambda i: (i, 0))] * 2,
        out_shape=[jax.ShapeDtypeStruct((N_NODES, HIDDEN_DIM), jnp.float32)] * 2,
    )(x, w1a, w1b, b1)


def _mlp_body(g_ref, e_ref, w1c_ref, w2_ref, b2_ref, w3_ref, b3_ref, out_ref):
    f32 = jnp.float32
    bf = jnp.bfloat16
    h = g_ref[...] + lax.dot_general(
        e_ref[...], w1c_ref[...], (((0,), (0,)), ((), ())),
        preferred_element_type=f32)
    h = jnp.maximum(h, 0.0).astype(bf)
    h = jnp.maximum(
        jnp.dot(h, w2_ref[...], preferred_element_type=f32) + b2_ref[...],
        0.0).astype(bf)
    out_ref[...] = jnp.dot(h, w3_ref[...], preferred_element_type=f32) + b3_ref[...]


def _run_mlp(g, e, w1c, w2, b2, w3, b3, block_e):
    grid = N_EDGES // block_e
    full = lambda s: pl.BlockSpec(s, lambda i: (0, 0))
    return pl.pallas_call(
        _mlp_body,
        grid=(grid,),
        in_specs=[
            pl.BlockSpec((block_e, NODE_DIM), lambda i: (i, 0)),
            pl.BlockSpec((EDGE_DIM, block_e), lambda i: (0, i)),
            full((EDGE_DIM, HIDDEN_DIM)),
            full((HIDDEN_DIM, HIDDEN_DIM)),
            full((1, HIDDEN_DIM)),
            full((HIDDEN_DIM, NODE_DIM)),
            full((1, NODE_DIM)),
        ],
        out_specs=pl.BlockSpec((block_e, NODE_DIM), lambda i: (i, 0)),
        out_shape=jax.ShapeDtypeStruct((N_EDGES, NODE_DIM), jnp.float32),
    )(g, e, w1c, w2, b2, w3, b3)


def _update_body(x_ref, p_ref, u1a_ref, u1b_ref, b1_ref, w2_ref, b2_ref,
                 gamma_ref, beta_ref, out_ref):
    f32 = jnp.float32
    xv = x_ref[...]
    aggr = p_ref[0] + p_ref[1]
    h = jnp.dot(xv, u1a_ref[...], preferred_element_type=f32)
    h += jnp.dot(aggr, u1b_ref[...], preferred_element_type=f32)
    h = jnp.maximum(h + b1_ref[...], 0.0)
    o = jnp.dot(h, w2_ref[...], preferred_element_type=f32) + b2_ref[...]
    y = xv + o
    mu = jnp.mean(y, axis=-1, keepdims=True)
    var = jnp.mean((y - mu) ** 2, axis=-1, keepdims=True)
    out_ref[...] = (y - mu) * lax.rsqrt(var + 1e-5) * gamma_ref[...] + beta_ref[...]


def _run_update(x, partials, u1a, u1b, b1, w2, b2, gamma, beta, block_n):
    grid = N_NODES // block_n
    full = lambda s: pl.BlockSpec(s, lambda i: (0,) * len(s))
    return pl.pallas_call(
        _update_body,
        grid=(grid,),
        in_specs=[
            pl.BlockSpec((block_n, NODE_DIM), lambda i: (i, 0)),
            pl.BlockSpec((NC, block_n, NODE_DIM), lambda i: (0, i, 0)),
            full((NODE_DIM, HIDDEN_DIM)),
            full((NODE_DIM, HIDDEN_DIM)),
            full((1, HIDDEN_DIM)),
            full((HIDDEN_DIM, NODE_DIM)),
            full((1, NODE_DIM)),
            full((1, NODE_DIM)),
            full((1, NODE_DIM)),
        ],
        out_specs=pl.BlockSpec((block_n, NODE_DIM), lambda i: (i, 0)),
        out_shape=jax.ShapeDtypeStruct((N_NODES, NODE_DIM), jnp.float32),
    )(x, partials, u1a, u1b, b1, w2, b2, gamma, beta)


def kernel(x, edge_index, edge_attr, m_w1, m_b1, m_w2, m_b2, m_w3, m_b3,
           u_w1, u_b1, u_w2, u_b2, gamma, beta):
    src = edge_index[0].astype(jnp.int32)
    dst = edge_index[1].astype(jnp.int32)

    gather_k, scatter_k = _sc_kernels()
    bf = jnp.bfloat16
    w1a = m_w1[:NODE_DIM].astype(bf)
    w1b = m_w1[NODE_DIM:2 * NODE_DIM].astype(bf)
    w1c = m_w1[2 * NODE_DIM:].astype(bf)
    xa, xb = _run_pre(x, w1a, w1b, m_b1.reshape(1, -1), block_n=1000)
    g = gather_k(xa, xb, src, dst)

    edge_attr = edge_attr.T.astype(bf)
    msg = _run_mlp(g, edge_attr, w1c, m_w2.astype(bf), m_b2.reshape(1, -1),
                   m_w3.astype(bf), m_b3.reshape(1, -1), block_e=2560)

    zeros = jnp.zeros((N_NODES, NODE_DIM), jnp.float32)
    partials = scatter_k(msg, dst, zeros)

    u1a = u_w1[:NODE_DIM]
    u1b = u_w1[NODE_DIM:]
    return _run_update(x, partials, u1a, u1b, u_b1.reshape(1, -1),
                       u_w2, u_b2.reshape(1, -1),
                       gamma.reshape(1, -1), beta.reshape(1, -1), block_n=1000)
